# initial kernel scaffold (unmeasured)
import jax
import jax.numpy as jnp
from jax import lax
from jax.experimental import pallas as pl
from jax.experimental.pallas import tpu as pltpu

N_DEV = 8
N_EXP = 32
E_LOC = N_EXP // N_DEV

_sem_signal = getattr(pl, "semaphore_signal", None) or pltpu.semaphore_signal
_sem_wait = getattr(pl, "semaphore_wait", None) or pltpu.semaphore_wait
_CompilerParams = getattr(pltpu, "CompilerParams", None) or pltpu.TPUCompilerParams


def kernel(x, router_W, route_idx, expert_W):
    n_tok, d_model = x.shape
    e_loc, _, d_ff = expert_W.shape

    def body(x_ref, rw_ref, idx_ref, ew_ref, out_ref,
             comm_ref, send_sems, recv_sems):
        my = lax.axis_index("i")
        left = lax.rem(my + (N_DEV - 1), N_DEV)
        right = lax.rem(my + 1, N_DEV)

        barrier_sem = pltpu.get_barrier_semaphore()
        for nbr in (left, right):
            _sem_signal(barrier_sem, inc=1, device_id=(nbr,),
                        device_id_type=pl.DeviceIdType.MESH)
        _sem_wait(barrier_sem, 2)

        xs = x_ref[:, :]
        scores = jnp.dot(xs, rw_ref[:, :],
                         preferred_element_type=jnp.float32)
        e0 = idx_ref[:, 0:1]
        e1 = idx_ref[:, 1:2]
        lanes = lax.broadcasted_iota(jnp.int32, (n_tok, N_EXP), 1)
        s0 = jnp.sum(jnp.where(lanes == e0, scores, 0.0), axis=-1,
                     keepdims=True)
        s1 = jnp.sum(jnp.where(lanes == e1, scores, 0.0), axis=-1,
                     keepdims=True)
        g0 = 1.0 / (1.0 + jnp.exp(s1 - s0))
        g1 = 1.0 - g0

        x_bf = xs.astype(jnp.bfloat16)

        comm_ref[0] = ew_ref[...].astype(jnp.bfloat16)

        def block_contrib(slot, src):
            base = src * E_LOC
            acc = None
            for j in range(E_LOC):
                e_id = base + j
                yj = jnp.dot(x_bf, comm_ref[slot, j],
                             preferred_element_type=jnp.float32)
                gate = (jnp.where(e0 == e_id, g0, 0.0)
                        + jnp.where(e1 == e_id, g1, 0.0))
                acc = gate * yj if acc is None else acc + gate * yj
            return acc

        out_ref[:, :] = block_contrib(0, my)

        for h in range(1, N_DEV):
            rdma = pltpu.make_async_remote_copy(
                src_ref=comm_ref.at[h - 1],
                dst_ref=comm_ref.at[h],
                send_sem=send_sems.at[h - 1],
                recv_sem=recv_sems.at[h - 1],
                device_id=(right,),
                device_id_type=pl.DeviceIdType.MESH,
            )
            rdma.start()
            rdma.wait()
            src = lax.rem(my + (N_DEV - h), N_DEV)
            out_ref[:, :] += block_contrib(h, src)

    out_shape = jax.ShapeDtypeStruct((n_tok, d_ff), jnp.float32)
    return pl.pallas_call(
        body,
        out_shape=out_shape,
        in_specs=[pl.BlockSpec(memory_space=pltpu.VMEM)] * 4,
        out_specs=pl.BlockSpec(memory_space=pltpu.VMEM),
        scratch_shapes=[
            pltpu.VMEM((N_DEV, e_loc, d_model, d_ff), jnp.bfloat16),
            pltpu.SemaphoreType.DMA((N_DEV - 1,)),
            pltpu.SemaphoreType.DMA((N_DEV - 1,)),
        ],
        compiler_params=_CompilerParams(collective_id=0),
    )(x, router_W, route_idx, expert_W)


# baseline (device time: 367209 ns/iter reference)
import jax
import jax.numpy as jnp
from jax import lax
from jax.experimental import pallas as pl
from jax.experimental.pallas import tpu as pltpu

N_DEV = 8
N_EXP = 32
E_LOC = N_EXP // N_DEV
N_SLOT = N_DEV - 1

_sem_signal = getattr(pl, "semaphore_signal", None) or pltpu.semaphore_signal
_sem_wait = getattr(pl, "semaphore_wait", None) or pltpu.semaphore_wait
_CompilerParams = getattr(pltpu, "CompilerParams", None) or pltpu.TPUCompilerParams


def kernel(x, router_W, route_idx, expert_W):
    n_tok, d_model = x.shape
    e_loc, _, d_ff = expert_W.shape
    k_dim = e_loc * d_model

    x_bf = x.astype(jnp.bfloat16)
    rw_bf = router_W.astype(jnp.bfloat16)
    ew_bf = expert_W.astype(jnp.bfloat16).reshape(k_dim, d_ff)

    def body(x_ref, rw_ref, idx_ref, ew_ref, out_ref,
             comm_ref, xg_ref, send_sems, recv_sems):
        my = lax.axis_index("i")
        left = lax.rem(my + (N_DEV - 1), N_DEV)
        right = lax.rem(my + 1, N_DEV)

        barrier_sem = pltpu.get_barrier_semaphore()
        for nbr in (left, right):
            _sem_signal(barrier_sem, inc=1, device_id=(nbr,),
                        device_id_type=pl.DeviceIdType.MESH)
        _sem_wait(barrier_sem, 2)

        xv = x_ref[:, :]
        scores = jnp.dot(xv, rw_ref[:, :],
                         preferred_element_type=jnp.float32)
        e0 = idx_ref[:, 0:1]
        e1 = idx_ref[:, 1:2]
        lanes = lax.broadcasted_iota(jnp.int32, (n_tok, N_EXP), 1)
        s0 = jnp.sum(jnp.where(lanes == e0, scores, 0.0), axis=-1,
                     keepdims=True)
        s1 = jnp.sum(jnp.where(lanes == e1, scores, 0.0), axis=-1,
                     keepdims=True)
        g0 = 1.0 / (1.0 + jnp.exp(s1 - s0))
        g1 = 1.0 - g0

        def block_contrib(w_2d_ref, src, first):
            base = src * E_LOC
            for j in range(E_LOC):
                e_id = base + j
                gate = (jnp.where(e0 == e_id, g0, 0.0)
                        + jnp.where(e1 == e_id, g1, 0.0))
                xg_ref[:, j * d_model:(j + 1) * d_model] = (
                    xv * gate.astype(jnp.bfloat16))
            y = jnp.dot(xg_ref[:, :], w_2d_ref[:, :],
                        preferred_element_type=jnp.float32)
            if first:
                out_ref[:, :] = y.astype(jnp.bfloat16)
            else:
                out_ref[:, :] += y.astype(jnp.bfloat16)

        for h in range(1, N_DEV):
            src_ref = ew_ref if h == 1 else comm_ref.at[h - 2]
            rdma = pltpu.make_async_remote_copy(
                src_ref=src_ref,
                dst_ref=comm_ref.at[h - 1],
                send_sem=send_sems.at[h - 1],
                recv_sem=recv_sems.at[h - 1],
                device_id=(right,),
                device_id_type=pl.DeviceIdType.MESH,
            )
            rdma.start()
            if h == 1:
                block_contrib(ew_ref, my, first=True)
            else:
                block_contrib(comm_ref.at[h - 2],
                              lax.rem(my + (N_DEV - (h - 1)), N_DEV),
                              first=False)
            rdma.wait()

        block_contrib(comm_ref.at[N_DEV - 2], lax.rem(my + 1, N_DEV),
                      first=False)

    out_shape = jax.ShapeDtypeStruct((n_tok, d_ff), jnp.bfloat16)
    out_bf = pl.pallas_call(
        body,
        out_shape=out_shape,
        in_specs=[pl.BlockSpec(memory_space=pltpu.VMEM)] * 4,
        out_specs=pl.BlockSpec(memory_space=pltpu.VMEM),
        scratch_shapes=[
            pltpu.VMEM((N_SLOT, k_dim, d_ff), jnp.bfloat16),
            pltpu.VMEM((n_tok, k_dim), jnp.bfloat16),
            pltpu.SemaphoreType.DMA((N_SLOT,)),
            pltpu.SemaphoreType.DMA((N_SLOT,)),
        ],
        compiler_params=_CompilerParams(
            collective_id=0, vmem_limit_bytes=100 * 1024 * 1024
        ),
    )(x_bf, rw_bf, route_idx, ew_bf)
    return out_bf.astype(jnp.float32)


# device time: 214519 ns/iter; 1.7118x vs baseline; 1.7118x over previous
import jax
import jax.numpy as jnp
from jax import lax
from jax.experimental import pallas as pl
from jax.experimental.pallas import tpu as pltpu

N_DEV = 8
N_EXP = 32
E_LOC = N_EXP // N_DEV
E_HALF = E_LOC // 2
N_SLOT = N_DEV - 1

_sem_signal = getattr(pl, "semaphore_signal", None) or pltpu.semaphore_signal
_sem_wait = getattr(pl, "semaphore_wait", None) or pltpu.semaphore_wait
_CompilerParams = getattr(pltpu, "CompilerParams", None) or pltpu.TPUCompilerParams


def kernel(x, router_W, route_idx, expert_W):
    n_tok, d_model = x.shape
    e_loc, _, d_ff = expert_W.shape
    k_half = E_HALF * d_model

    x_bf = x.astype(jnp.bfloat16)
    rw_bf = router_W.astype(jnp.bfloat16)
    ew2d = expert_W.astype(jnp.bfloat16).reshape(e_loc * d_model, d_ff)
    ew_r = ew2d[:k_half]
    ew_l = ew2d[k_half:]

    def body(x_ref, rw_ref, idx_ref, ewr_ref, ewl_ref, out_ref,
             commr_ref, comml_ref, xg_ref,
             sendr_sems, recvr_sems, sendl_sems, recvl_sems):
        my = lax.axis_index("i")
        left = lax.rem(my + (N_DEV - 1), N_DEV)
        right = lax.rem(my + 1, N_DEV)

        barrier_sem = pltpu.get_barrier_semaphore()
        for nbr in (left, right):
            _sem_signal(barrier_sem, inc=1, device_id=(nbr,),
                        device_id_type=pl.DeviceIdType.MESH)
        _sem_wait(barrier_sem, 2)

        xv = x_ref[:, :]
        scores = jnp.dot(xv, rw_ref[:, :],
                         preferred_element_type=jnp.float32)
        e0 = idx_ref[:, 0:1]
        e1 = idx_ref[:, 1:2]
        lanes = lax.broadcasted_iota(jnp.int32, (n_tok, N_EXP), 1)
        s0 = jnp.sum(jnp.where(lanes == e0, scores, 0.0), axis=-1,
                     keepdims=True)
        s1 = jnp.sum(jnp.where(lanes == e1, scores, 0.0), axis=-1,
                     keepdims=True)
        g0 = 1.0 / (1.0 + jnp.exp(s1 - s0))
        g1 = 1.0 - g0

        def half_contrib(w_ref, src, pair, first=False):
            base = src * E_LOC + pair * E_HALF
            for j in range(E_HALF):
                e_id = base + j
                gate = (jnp.where(e0 == e_id, g0, 0.0)
                        + jnp.where(e1 == e_id, g1, 0.0))
                xg_ref[:, j * d_model:(j + 1) * d_model] = (
                    xv * gate.astype(jnp.bfloat16))
            y = jnp.dot(xg_ref[:, :], w_ref[:, :],
                        preferred_element_type=jnp.float32)
            if first:
                out_ref[:, :] = y.astype(jnp.bfloat16)
            else:
                out_ref[:, :] += y.astype(jnp.bfloat16)

        for h in range(1, N_DEV):
            rdma_r = pltpu.make_async_remote_copy(
                src_ref=ewr_ref if h == 1 else commr_ref.at[h - 2],
                dst_ref=commr_ref.at[h - 1],
                send_sem=sendr_sems.at[h - 1],
                recv_sem=recvr_sems.at[h - 1],
                device_id=(right,),
                device_id_type=pl.DeviceIdType.MESH,
            )
            rdma_l = pltpu.make_async_remote_copy(
                src_ref=ewl_ref if h == 1 else comml_ref.at[h - 2],
                dst_ref=comml_ref.at[h - 1],
                send_sem=sendl_sems.at[h - 1],
                recv_sem=recvl_sems.at[h - 1],
                device_id=(left,),
                device_id_type=pl.DeviceIdType.MESH,
            )
            rdma_r.start()
            rdma_l.start()
            if h == 1:
                half_contrib(ewr_ref, my, 0, first=True)
                half_contrib(ewl_ref, my, 1)
            else:
                half_contrib(commr_ref.at[h - 2],
                             lax.rem(my + (N_DEV - (h - 1)), N_DEV), 0)
                half_contrib(comml_ref.at[h - 2],
                             lax.rem(my + (h - 1), N_DEV), 1)
            rdma_r.wait()
            rdma_l.wait()

        half_contrib(commr_ref.at[N_DEV - 2], lax.rem(my + 1, N_DEV), 0)
        half_contrib(comml_ref.at[N_DEV - 2],
                     lax.rem(my + (N_DEV - 1), N_DEV), 1)

    out_shape = jax.ShapeDtypeStruct((n_tok, d_ff), jnp.bfloat16)
    out_bf = pl.pallas_call(
        body,
        out_shape=out_shape,
        in_specs=[pl.BlockSpec(memory_space=pltpu.VMEM)] * 5,
        out_specs=pl.BlockSpec(memory_space=pltpu.VMEM),
        scratch_shapes=[
            pltpu.VMEM((N_SLOT, k_half, d_ff), jnp.bfloat16),
            pltpu.VMEM((N_SLOT, k_half, d_ff), jnp.bfloat16),
            pltpu.VMEM((n_tok, k_half), jnp.bfloat16),
            pltpu.SemaphoreType.DMA((N_SLOT,)),
            pltpu.SemaphoreType.DMA((N_SLOT,)),
            pltpu.SemaphoreType.DMA((N_SLOT,)),
            pltpu.SemaphoreType.DMA((N_SLOT,)),
        ],
        compiler_params=_CompilerParams(
            collective_id=0, vmem_limit_bytes=100 * 1024 * 1024
        ),
    )(x_bf, rw_bf, route_idx, ew_r, ew_l)
    return out_bf.astype(jnp.float32)


# device time: 213095 ns/iter; 1.7232x vs baseline; 1.0067x over previous
import jax
import jax.numpy as jnp
from jax import lax
from jax.experimental import pallas as pl
from jax.experimental.pallas import tpu as pltpu

N_DEV = 8
N_EXP = 32
E_LOC = N_EXP // N_DEV
E_HALF = E_LOC // 2
N_SLOT = N_DEV - 1

_sem_signal = getattr(pl, "semaphore_signal", None) or pltpu.semaphore_signal
_sem_wait = getattr(pl, "semaphore_wait", None) or pltpu.semaphore_wait
_CompilerParams = getattr(pltpu, "CompilerParams", None) or pltpu.TPUCompilerParams


def kernel(x, router_W, route_idx, expert_W):
    n_tok, d_model = x.shape
    e_loc, _, d_ff = expert_W.shape
    k_half = E_HALF * d_model

    x_bf = x.astype(jnp.bfloat16)
    rw_bf = router_W.astype(jnp.bfloat16)
    ew2d = expert_W.astype(jnp.bfloat16).reshape(e_loc * d_model, d_ff)
    ew_r = ew2d[:k_half]
    ew_l = ew2d[k_half:]

    def body(x_ref, rw_ref, idx_ref, ewr_ref, ewl_ref, out_ref,
             commr_ref, comml_ref, xg_ref,
             sendr_sems, recvr_sems, sendl_sems, recvl_sems):
        my = lax.axis_index("i")
        left = lax.rem(my + (N_DEV - 1), N_DEV)
        right = lax.rem(my + 1, N_DEV)

        barrier_sem = pltpu.get_barrier_semaphore()
        for nbr in (left, right):
            _sem_signal(barrier_sem, inc=1, device_id=(nbr,),
                        device_id_type=pl.DeviceIdType.MESH)
        _sem_wait(barrier_sem, 2)

        def make_hop(h):
            rdma_r = pltpu.make_async_remote_copy(
                src_ref=ewr_ref if h == 1 else commr_ref.at[h - 2],
                dst_ref=commr_ref.at[h - 1],
                send_sem=sendr_sems.at[h - 1],
                recv_sem=recvr_sems.at[h - 1],
                device_id=(right,),
                device_id_type=pl.DeviceIdType.MESH,
            )
            rdma_l = pltpu.make_async_remote_copy(
                src_ref=ewl_ref if h == 1 else comml_ref.at[h - 2],
                dst_ref=comml_ref.at[h - 1],
                send_sem=sendl_sems.at[h - 1],
                recv_sem=recvl_sems.at[h - 1],
                device_id=(left,),
                device_id_type=pl.DeviceIdType.MESH,
            )
            return rdma_r, rdma_l

        hops = [make_hop(1)]
        hops[0][0].start()
        hops[0][1].start()

        xv = x_ref[:, :]
        scores = jnp.dot(xv, rw_ref[:, :],
                         preferred_element_type=jnp.float32)
        e0 = idx_ref[:, 0:1]
        e1 = idx_ref[:, 1:2]
        lanes = lax.broadcasted_iota(jnp.int32, (n_tok, N_EXP), 1)
        s0 = jnp.sum(jnp.where(lanes == e0, scores, 0.0), axis=-1,
                     keepdims=True)
        s1 = jnp.sum(jnp.where(lanes == e1, scores, 0.0), axis=-1,
                     keepdims=True)
        g0 = 1.0 / (1.0 + jnp.exp(s1 - s0))
        g1 = 1.0 - g0

        def half_contrib(w_ref, src, pair, first=False):
            base = src * E_LOC + pair * E_HALF
            for j in range(E_HALF):
                e_id = base + j
                gate = (jnp.where(e0 == e_id, g0, 0.0)
                        + jnp.where(e1 == e_id, g1, 0.0))
                xg_ref[:, j * d_model:(j + 1) * d_model] = (
                    xv * gate.astype(jnp.bfloat16))
            y = jnp.dot(xg_ref[:, :], w_ref[:, :],
                        preferred_element_type=jnp.float32)
            if first:
                out_ref[:, :] = y.astype(jnp.bfloat16)
            else:
                out_ref[:, :] += y.astype(jnp.bfloat16)

        half_contrib(ewr_ref, my, 0, first=True)
        half_contrib(ewl_ref, my, 1)

        for h in range(2, N_DEV):
            prev_r, prev_l = hops[-1]
            prev_r.wait_recv()
            prev_l.wait_recv()
            rdma_r, rdma_l = make_hop(h)
            rdma_r.start()
            rdma_l.start()
            hops.append((rdma_r, rdma_l))
            half_contrib(commr_ref.at[h - 2],
                         lax.rem(my + (N_DEV - (h - 1)), N_DEV), 0)
            half_contrib(comml_ref.at[h - 2],
                         lax.rem(my + (h - 1), N_DEV), 1)

        last_r, last_l = hops[-1]
        last_r.wait_recv()
        half_contrib(commr_ref.at[N_DEV - 2], lax.rem(my + 1, N_DEV), 0)
        last_l.wait_recv()
        half_contrib(comml_ref.at[N_DEV - 2],
                     lax.rem(my + (N_DEV - 1), N_DEV), 1)

        for rdma_r, rdma_l in hops:
            rdma_r.wait_send()
            rdma_l.wait_send()

    out_shape = jax.ShapeDtypeStruct((n_tok, d_ff), jnp.bfloat16)
    out_bf = pl.pallas_call(
        body,
        out_shape=out_shape,
        in_specs=[pl.BlockSpec(memory_space=pltpu.VMEM)] * 5,
        out_specs=pl.BlockSpec(memory_space=pltpu.VMEM),
        scratch_shapes=[
            pltpu.VMEM((N_SLOT, k_half, d_ff), jnp.bfloat16),
            pltpu.VMEM((N_SLOT, k_half, d_ff), jnp.bfloat16),
            pltpu.VMEM((n_tok, k_half), jnp.bfloat16),
            pltpu.SemaphoreType.DMA((N_SLOT,)),
            pltpu.SemaphoreType.DMA((N_SLOT,)),
            pltpu.SemaphoreType.DMA((N_SLOT,)),
            pltpu.SemaphoreType.DMA((N_SLOT,)),
        ],
        compiler_params=_CompilerParams(
            collective_id=0, vmem_limit_bytes=100 * 1024 * 1024
        ),
    )(x_bf, rw_bf, route_idx, ew_r, ew_l)
    return out_bf.astype(jnp.float32)


# device time: 212234 ns/iter; 1.7302x vs baseline; 1.0041x over previous
import jax
import jax.numpy as jnp
from jax import lax
from jax.experimental import pallas as pl
from jax.experimental.pallas import tpu as pltpu

N_DEV = 8
N_EXP = 32
E_LOC = N_EXP // N_DEV
E_HALF = E_LOC // 2
N_SLOT = N_DEV - 1

_sem_signal = getattr(pl, "semaphore_signal", None) or pltpu.semaphore_signal
_sem_wait = getattr(pl, "semaphore_wait", None) or pltpu.semaphore_wait
_CompilerParams = getattr(pltpu, "CompilerParams", None) or pltpu.TPUCompilerParams


def kernel(x, router_W, route_idx, expert_W):
    n_tok, d_model = x.shape
    e_loc, _, d_ff = expert_W.shape
    k_half = E_HALF * d_model

    x_bf = x.astype(jnp.bfloat16)
    rw_bf = router_W.astype(jnp.bfloat16)
    ew2d = expert_W.astype(jnp.bfloat16).reshape(e_loc * d_model, d_ff)
    ew_r = ew2d[:k_half]
    ew_l = ew2d[k_half:]

    def body(x_ref, rw_ref, idx_ref, ewr_ref, ewl_ref, out_ref,
             commr_ref, comml_ref, xg_ref,
             sendr_sems, recvr_sems, sendl_sems, recvl_sems):
        my = lax.axis_index("i")
        left = lax.rem(my + (N_DEV - 1), N_DEV)
        right = lax.rem(my + 1, N_DEV)

        barrier_sem = pltpu.get_barrier_semaphore()
        for nbr in (left, right):
            _sem_signal(barrier_sem, inc=1, device_id=(nbr,),
                        device_id_type=pl.DeviceIdType.MESH)
        _sem_wait(barrier_sem, 2)

        def make_hop(h):
            rdma_r = pltpu.make_async_remote_copy(
                src_ref=ewr_ref if h == 1 else commr_ref.at[h - 2],
                dst_ref=commr_ref.at[h - 1],
                send_sem=sendr_sems.at[h - 1],
                recv_sem=recvr_sems.at[h - 1],
                device_id=(right,),
                device_id_type=pl.DeviceIdType.MESH,
            )
            rdma_l = pltpu.make_async_remote_copy(
                src_ref=ewl_ref if h == 1 else comml_ref.at[h - 2],
                dst_ref=comml_ref.at[h - 1],
                send_sem=sendl_sems.at[h - 1],
                recv_sem=recvl_sems.at[h - 1],
                device_id=(left,),
                device_id_type=pl.DeviceIdType.MESH,
            )
            return rdma_r, rdma_l

        hops = [make_hop(1)]
        hops[0][0].start()
        hops[0][1].start()

        xv = x_ref[:, :]
        scores = jnp.dot(xv, rw_ref[:, :],
                         preferred_element_type=jnp.float32)
        e0 = idx_ref[:, 0:1]
        e1 = idx_ref[:, 1:2]
        lanes = lax.broadcasted_iota(jnp.int32, (n_tok, N_EXP), 1)
        s0 = jnp.sum(jnp.where(lanes == e0, scores, 0.0), axis=-1,
                     keepdims=True)
        s1 = jnp.sum(jnp.where(lanes == e1, scores, 0.0), axis=-1,
                     keepdims=True)
        g0 = 1.0 / (1.0 + jnp.exp(s1 - s0))
        g1 = 1.0 - g0

        def pair_contrib(wr_ref, wl_ref, src_r, src_l, first=False):
            y = None
            for w_ref, src, pair in ((wr_ref, src_r, 0), (wl_ref, src_l, 1)):
                base = src * E_LOC + pair * E_HALF
                for j in range(E_HALF):
                    e_id = base + j
                    gate = (jnp.where(e0 == e_id, g0, 0.0)
                            + jnp.where(e1 == e_id, g1, 0.0))
                    xg_ref[:, j * d_model:(j + 1) * d_model] = (
                        xv * gate.astype(jnp.bfloat16))
                d = jnp.dot(xg_ref[:, :], w_ref[:, :],
                            preferred_element_type=jnp.float32)
                y = d if y is None else y + d
            if first:
                out_ref[:, :] = y.astype(jnp.bfloat16)
            else:
                out_ref[:, :] += y.astype(jnp.bfloat16)

        pair_contrib(ewr_ref, ewl_ref, my, my, first=True)

        for h in range(2, N_DEV):
            prev_r, prev_l = hops[-1]
            prev_r.wait_recv()
            prev_l.wait_recv()
            rdma_r, rdma_l = make_hop(h)
            rdma_r.start()
            rdma_l.start()
            hops.append((rdma_r, rdma_l))
            pair_contrib(commr_ref.at[h - 2], comml_ref.at[h - 2],
                         lax.rem(my + (N_DEV - (h - 1)), N_DEV),
                         lax.rem(my + (h - 1), N_DEV))

        last_r, last_l = hops[-1]
        last_r.wait_recv()
        last_l.wait_recv()
        pair_contrib(commr_ref.at[N_DEV - 2], comml_ref.at[N_DEV - 2],
                     lax.rem(my + 1, N_DEV),
                     lax.rem(my + (N_DEV - 1), N_DEV))

        for rdma_r, rdma_l in hops:
            rdma_r.wait_send()
            rdma_l.wait_send()

    out_shape = jax.ShapeDtypeStruct((n_tok, d_ff), jnp.bfloat16)
    out_bf = pl.pallas_call(
        body,
        out_shape=out_shape,
        in_specs=[pl.BlockSpec(memory_space=pltpu.VMEM)] * 5,
        out_specs=pl.BlockSpec(memory_space=pltpu.VMEM),
        scratch_shapes=[
            pltpu.VMEM((N_SLOT, k_half, d_ff), jnp.bfloat16),
            pltpu.VMEM((N_SLOT, k_half, d_ff), jnp.bfloat16),
            pltpu.VMEM((n_tok, k_half), jnp.bfloat16),
            pltpu.SemaphoreType.DMA((N_SLOT,)),
            pltpu.SemaphoreType.DMA((N_SLOT,)),
            pltpu.SemaphoreType.DMA((N_SLOT,)),
            pltpu.SemaphoreType.DMA((N_SLOT,)),
        ],
        compiler_params=_CompilerParams(
            collective_id=0, vmem_limit_bytes=100 * 1024 * 1024
        ),
    )(x_bf, rw_bf, route_idx, ew_r, ew_l)
    return out_bf.astype(jnp.float32)
